# Initial kernel scaffold; baseline (speedup 1.0000x reference)
import jax
import jax.numpy as jnp
from jax import lax
from jax.experimental import pallas as pl
from jax.experimental.pallas import tpu as pltpu
from jax.experimental.pallas import tpu_sc as plsc

SEQ = 200
DIM = 32
BATCH = 4096
ROWS = BATCH * SEQ
NW = 32
ROWS_W = ROWS // NW
CHUNK = 1600
NCHUNK = ROWS_W // CHUNK
GSUB = 100
NGATH = CHUNK // GSUB
BPC = CHUNK // SEQ
SCALE = float(DIM) ** 0.5


def _body(idx_hbm, table_hbm, pos_hbm, out_hbm, idx_v, rows_v, pos_v, sem):
    cidx = lax.axis_index("c")
    sidx = lax.axis_index("s")
    wid = sidx * 2 + cidx

    pltpu.sync_copy(pos_hbm, pos_v)

    def chunk_body(c, carry):
        row0 = wid * ROWS_W + c * CHUNK
        irow0 = row0 // GSUB
        pltpu.sync_copy(idx_hbm.at[pl.ds(irow0, NGATH)], idx_v)

        copies = []
        for j in range(NGATH):
            copies.append(
                pltpu.async_copy(
                    table_hbm.at[idx_v.at[j]],
                    rows_v.at[pl.ds(j * GSUB, GSUB)],
                    sem,
                )
            )
        for cp in copies:
            cp.wait()

        def s_body(s, carry2):
            p0 = pos_v[s, pl.ds(0, 16)]
            p1 = pos_v[s, pl.ds(16, 16)]
            for bi in range(BPC):
                r = bi * SEQ + s
                rows_v[r, pl.ds(0, 16)] = rows_v[r, pl.ds(0, 16)] * SCALE + p0
                rows_v[r, pl.ds(16, 16)] = rows_v[r, pl.ds(16, 16)] * SCALE + p1
            return carry2

        lax.fori_loop(0, SEQ, s_body, 0)

        pltpu.sync_copy(rows_v, out_hbm.at[pl.ds(row0, CHUNK)])
        return carry

    lax.fori_loop(0, NCHUNK, chunk_body, 0)


@jax.jit
def kernel(inputs, token_table, pos_table):
    idx = inputs.reshape(ROWS // GSUB, GSUB).astype(jnp.int32)
    mesh = plsc.VectorSubcoreMesh(core_axis_name="c", subcore_axis_name="s")
    out = pl.kernel(
        _body,
        out_type=jax.ShapeDtypeStruct((ROWS, DIM), jnp.float32),
        mesh=mesh,
        scratch_types=[
            pltpu.VMEM((NGATH, GSUB), jnp.int32),
            pltpu.VMEM((CHUNK, DIM), jnp.float32),
            pltpu.VMEM((SEQ, DIM), jnp.float32),
            pltpu.SemaphoreType.DMA,
        ],
    )(idx, token_table, pos_table)
    return out.reshape(BATCH, SEQ, DIM)


# speedup vs baseline: 1.4253x; 1.4253x over previous
import jax
import jax.numpy as jnp
from jax import lax
from jax.experimental import pallas as pl
from jax.experimental.pallas import tpu as pltpu
from jax.experimental.pallas import tpu_sc as plsc

SEQ = 200
DIM = 32
BATCH = 4096
ROWS = BATCH * SEQ
NW = 32
ROWS_W = ROWS // NW
CHUNK = 1600
NCHUNK = ROWS_W // CHUNK
GSUB = 100
NGATH = CHUNK // GSUB
BPC = CHUNK // SEQ
SCALE = float(DIM) ** 0.5


def _body(idx_hbm, table_hbm, pos_hbm, out_hbm, idx_v, rows_v, pos_v, sem):
    cidx = lax.axis_index("c")
    sidx = lax.axis_index("s")
    wid = sidx * 2 + cidx

    pltpu.sync_copy(pos_hbm, pos_v)

    def chunk_body(c, carry):
        row0 = pl.multiple_of(wid * ROWS_W + c * CHUNK, 8)
        irow0 = pl.multiple_of(row0 // GSUB, 8)
        pltpu.sync_copy(idx_hbm.at[pl.ds(irow0, NGATH)], idx_v)

        copies = []
        for j in range(NGATH):
            copies.append(
                pltpu.async_copy(
                    table_hbm.at[idx_v.at[j]],
                    rows_v.at[pl.ds(j * GSUB, GSUB)],
                    sem,
                )
            )
        for cp in copies:
            cp.wait()

        def s_body(s, carry2):
            p0 = pos_v[s, pl.ds(0, 16)]
            p1 = pos_v[s, pl.ds(16, 16)]
            for bi in range(BPC):
                r = bi * SEQ + s
                rows_v[r, pl.ds(0, 16)] = rows_v[r, pl.ds(0, 16)] * SCALE + p0
                rows_v[r, pl.ds(16, 16)] = rows_v[r, pl.ds(16, 16)] * SCALE + p1
            return carry2

        lax.fori_loop(0, SEQ, s_body, 0)

        pltpu.sync_copy(rows_v, out_hbm.at[pl.ds(row0, CHUNK)])
        return carry

    lax.fori_loop(0, NCHUNK, chunk_body, 0)


@jax.jit
def kernel(inputs, token_table, pos_table):
    idx = inputs.reshape(ROWS // GSUB, GSUB).astype(jnp.int32)
    mesh = plsc.VectorSubcoreMesh(core_axis_name="c", subcore_axis_name="s")
    out = pl.kernel(
        _body,
        out_type=jax.ShapeDtypeStruct((ROWS, DIM), jnp.float32),
        mesh=mesh,
        scratch_types=[
            pltpu.VMEM((NGATH, GSUB), jnp.int32),
            pltpu.VMEM((CHUNK, DIM), jnp.float32),
            pltpu.VMEM((SEQ, DIM), jnp.float32),
            pltpu.SemaphoreType.DMA,
        ],
        compiler_params=pltpu.CompilerParams(use_tc_tiling_on_sc=False),
    )(idx, token_table, pos_table)
    return out.reshape(BATCH, SEQ, DIM)
